# baseline (speedup 1.0000x reference)
import jax
import jax.numpy as jnp
import numpy as np
from jax.experimental import pallas as pl
from jax.experimental.pallas import tpu as pltpu

NUM_ENC = 6
TEMP = 0.1
OMEGAS = (50.0, 60.0, 60.0, 70.0, 70.0, 50.0)
TILE = 2048

_SIN_C = (0.9999813524244627, -0.16662701666609445, 0.008309896430163907,
          -0.00019276207301946195, 2.1522576752281716e-06)
_INV2PI = 0.15915494309189535
_TWOPI = 6.283185307179586
_MAGIC = 1.5 * 2.0 ** 23


def _sin_reduced(t):
    k = jnp.floor(t * _INV2PI + 0.5)
    r = t - k * _TWOPI
    r2 = r * r
    p = _SIN_C[4]
    for i in range(3, -1, -1):
        p = p * r2 + _SIN_C[i]
    return p * r


def _fused_kernel(x_ref, RW1_ref, Rb1_ref, RW2_ref, Rb2_ref,
                  W0_ref, b0_ref, W1_ref, b1_ref, W2_ref, b2_ref,
                  W3_ref, b3_ref, W4_ref, b4_ref, W5_ref, b5_ref,
                  psc_ref, pph_ref, S_ref, out_ref):
    x = x_ref[...]
    xt = jnp.concatenate([x] * (2 * NUM_ENC), axis=-1)
    pe = _sin_reduced(xt * psc_ref[...] + pph_ref[...])
    enc = jnp.concatenate([x, pe], axis=-1)

    hr = jnp.dot(enc, RW1_ref[...], preferred_element_type=jnp.float32)
    hr = jnp.maximum(hr + Rb1_ref[...], 0.0)
    logits = jnp.dot(hr, RW2_ref[...], preferred_element_type=jnp.float32)
    logits = (logits + Rb2_ref[...]) * (1.0 / TEMP)
    m = jnp.max(logits, axis=-1, keepdims=True)
    ex = jnp.exp(logits - m)
    w = ex / jnp.sum(ex, axis=-1, keepdims=True)

    z0 = jnp.dot(enc, W0_ref[...], preferred_element_type=jnp.float32)
    h0 = _sin_reduced(z0 * OMEGAS[0] + b0_ref[...])
    hs = [h0[:, 256 * e:256 * (e + 1)] for e in range(6)]

    for li, W_ref, b_ref in ((1, W1_ref, b1_ref), (2, W2_ref, b2_ref),
                             (3, W3_ref, b3_ref), (4, W4_ref, b4_ref)):
        zs = [jnp.dot(hs[e], W_ref[e], preferred_element_type=jnp.float32)
              for e in range(6)]
        hs = [_sin_reduced(zs[e] * OMEGAS[li] + b_ref[e]) for e in range(6)]

    hcat = jnp.concatenate(hs, axis=-1)
    z18 = jnp.dot(hcat, W5_ref[...], preferred_element_type=jnp.float32)
    y18 = _sin_reduced(z18 * OMEGAS[5] + b5_ref[...])
    w3 = jnp.concatenate([w, w, w], axis=-1)
    out_ref[...] = jnp.dot(y18 * w3, S_ref[...],
                           preferred_element_type=jnp.float32)


def kernel(x, RW1, Rb1, RW2, Rb2, EW0, Eb0, EW1, Eb1, EW2, Eb2,
           EW3, Eb3, EW4, Eb4, EW5, Eb5):
    n = x.shape[0]
    tile = TILE if n % TILE == 0 else n
    grid = n // tile

    Rb1_2d = Rb1.reshape(1, -1)
    Rb2_2d = Rb2.reshape(1, -1)
    W0c = EW0.transpose(1, 0, 2).reshape(EW0.shape[1], -1)
    b0c = Eb0.reshape(1, -1) * OMEGAS[0]
    Wf = [EW1, EW2, EW3, EW4]
    bf = [(Eb * om).reshape(Eb.shape[0], 1, Eb.shape[1])
          for Eb, om in zip((Eb1, Eb2, Eb3, Eb4), OMEGAS[1:5])]
    W5bd = (EW5[:, :, :, None] * jnp.eye(6, dtype=jnp.float32)[:, None, None, :])
    W5bd = W5bd.reshape(6 * EW5.shape[1], 3 * 6)
    b5p = (Eb5 * OMEGAS[5]).T.reshape(1, -1)
    Ssel = np.zeros((18, 3), np.float32)
    for o in range(3):
        Ssel[6 * o:6 * o + 6, o] = 1.0
    Ssel = jnp.asarray(Ssel)

    psc = np.zeros((1, 2 * NUM_ENC * 2), np.float32)
    pph = np.zeros((1, 2 * NUM_ENC * 2), np.float32)
    for i in range(NUM_ENC):
        s = (2.0 ** i) * np.pi
        psc[0, 4 * i:4 * i + 4] = s
        pph[0, 4 * i + 2:4 * i + 4] = np.pi / 2
    psc = jnp.asarray(psc)
    pph = jnp.asarray(pph)

    def full(a):
        nd = a.ndim
        return pl.BlockSpec(a.shape, lambda i, _nd=nd: (0,) * _nd)

    operands = (RW1, Rb1_2d, RW2, Rb2_2d, W0c, b0c,
                Wf[0], bf[0], Wf[1], bf[1], Wf[2], bf[2],
                Wf[3], bf[3], W5bd, b5p, psc, pph, Ssel)
    in_specs = [pl.BlockSpec((tile, 2), lambda i: (i, 0))]
    in_specs += [full(a) for a in operands]

    out = pl.pallas_call(
        _fused_kernel,
        grid=(grid,),
        in_specs=in_specs,
        out_specs=pl.BlockSpec((tile, 3), lambda i: (i, 0)),
        compiler_params=pltpu.CompilerParams(
            dimension_semantics=("parallel",),
            vmem_limit_bytes=128 * 1024 * 1024),
        out_shape=jax.ShapeDtypeStruct((n, 3), jnp.float32),
    )(x, *operands)
    return out


# speedup vs baseline: 1.1858x; 1.1858x over previous
import jax
import jax.numpy as jnp
import numpy as np
from jax.experimental import pallas as pl
from jax.experimental.pallas import tpu as pltpu

NUM_ENC = 6
TEMP = 0.1
OMEGAS = (50.0, 60.0, 60.0, 70.0, 70.0, 50.0)
TILE = 1024

_SIN_C = (0.9999813524244627, -0.16662701666609445, 0.008309896430163907,
          -0.00019276207301946195, 2.1522576752281716e-06)
_INV2PI = 0.15915494309189535
_TWOPI = 6.283185307179586
_MAGIC = 1.5 * 2.0 ** 23


def _sin_reduced(t):
    k = jnp.floor(t * _INV2PI + 0.5)
    r = t - k * _TWOPI
    r2 = r * r
    p = _SIN_C[4]
    for i in range(3, -1, -1):
        p = p * r2 + _SIN_C[i]
    return p * r


def _fused_kernel(x_ref, RW1_ref, Rb1_ref, RW2_ref, Rb2_ref,
                  W0_ref, b0_ref, W1_ref, b1_ref, W2_ref, b2_ref,
                  W3_ref, b3_ref, W4_ref, b4_ref, W5_ref, b5_ref,
                  psc_ref, pph_ref, S_ref, out_ref):
    x = x_ref[...]
    xt = jnp.concatenate([x] * (2 * NUM_ENC), axis=-1)
    pe = _sin_reduced(xt * psc_ref[...] + pph_ref[...])
    enc = jnp.concatenate([x, pe], axis=-1)

    hr = jnp.dot(enc, RW1_ref[...], preferred_element_type=jnp.float32)
    hr = jnp.maximum(hr + Rb1_ref[...], 0.0)
    logits = jnp.dot(hr, RW2_ref[...], preferred_element_type=jnp.float32)
    logits = (logits + Rb2_ref[...]) * (1.0 / TEMP)
    m = jnp.max(logits, axis=-1, keepdims=True)
    ex = jnp.exp(logits - m)
    w = ex / jnp.sum(ex, axis=-1, keepdims=True)

    z0 = jnp.dot(enc, W0_ref[...], preferred_element_type=jnp.float32)
    h0 = _sin_reduced(z0 * OMEGAS[0] + b0_ref[...])
    hs = [h0[:, 256 * e:256 * (e + 1)] for e in range(6)]

    for li, W_ref, b_ref in ((1, W1_ref, b1_ref), (2, W2_ref, b2_ref),
                             (3, W3_ref, b3_ref), (4, W4_ref, b4_ref)):
        zs = [jnp.dot(hs[e], W_ref[e], preferred_element_type=jnp.float32)
              for e in range(6)]
        hs = [_sin_reduced(zs[e] * OMEGAS[li] + b_ref[e]) for e in range(6)]

    hcat = jnp.concatenate(hs, axis=-1)
    z18 = jnp.dot(hcat, W5_ref[...], preferred_element_type=jnp.float32)
    y18 = _sin_reduced(z18 * OMEGAS[5] + b5_ref[...])
    w3 = jnp.concatenate([w, w, w], axis=-1)
    out_ref[...] = jnp.dot(y18 * w3, S_ref[...],
                           preferred_element_type=jnp.float32)


def kernel(x, RW1, Rb1, RW2, Rb2, EW0, Eb0, EW1, Eb1, EW2, Eb2,
           EW3, Eb3, EW4, Eb4, EW5, Eb5):
    n = x.shape[0]
    tile = TILE if n % TILE == 0 else n
    grid = n // tile

    Rb1_2d = Rb1.reshape(1, -1)
    Rb2_2d = Rb2.reshape(1, -1)
    W0c = EW0.transpose(1, 0, 2).reshape(EW0.shape[1], -1)
    b0c = Eb0.reshape(1, -1) * OMEGAS[0]
    Wf = [EW1, EW2, EW3, EW4]
    bf = [(Eb * om).reshape(Eb.shape[0], 1, Eb.shape[1])
          for Eb, om in zip((Eb1, Eb2, Eb3, Eb4), OMEGAS[1:5])]
    W5bd = (EW5[:, :, :, None] * jnp.eye(6, dtype=jnp.float32)[:, None, None, :])
    W5bd = W5bd.reshape(6 * EW5.shape[1], 3 * 6)
    b5p = (Eb5 * OMEGAS[5]).T.reshape(1, -1)
    Ssel = np.zeros((18, 3), np.float32)
    for o in range(3):
        Ssel[6 * o:6 * o + 6, o] = 1.0
    Ssel = jnp.asarray(Ssel)

    psc = np.zeros((1, 2 * NUM_ENC * 2), np.float32)
    pph = np.zeros((1, 2 * NUM_ENC * 2), np.float32)
    for i in range(NUM_ENC):
        s = (2.0 ** i) * np.pi
        psc[0, 4 * i:4 * i + 4] = s
        pph[0, 4 * i + 2:4 * i + 4] = np.pi / 2
    psc = jnp.asarray(psc)
    pph = jnp.asarray(pph)

    def full(a):
        nd = a.ndim
        return pl.BlockSpec(a.shape, lambda i, _nd=nd: (0,) * _nd)

    operands = (RW1, Rb1_2d, RW2, Rb2_2d, W0c, b0c,
                Wf[0], bf[0], Wf[1], bf[1], Wf[2], bf[2],
                Wf[3], bf[3], W5bd, b5p, psc, pph, Ssel)
    in_specs = [pl.BlockSpec((tile, 2), lambda i: (i, 0))]
    in_specs += [full(a) for a in operands]

    out = pl.pallas_call(
        _fused_kernel,
        grid=(grid,),
        in_specs=in_specs,
        out_specs=pl.BlockSpec((tile, 3), lambda i: (i, 0)),
        compiler_params=pltpu.CompilerParams(
            dimension_semantics=("arbitrary",),
            vmem_limit_bytes=128 * 1024 * 1024),
        out_shape=jax.ShapeDtypeStruct((n, 3), jnp.float32),
    )(x, *operands)
    return out


# speedup vs baseline: 1.2189x; 1.0279x over previous
import jax
import jax.numpy as jnp
import numpy as np
from jax.experimental import pallas as pl
from jax.experimental.pallas import tpu as pltpu

NUM_ENC = 6
TEMP = 0.1
OMEGAS = (50.0, 60.0, 60.0, 70.0, 70.0, 50.0)
TILE = 1024

_SIN_C = (0.9999813524244627, -0.16662701666609445, 0.008309896430163907,
          -0.00019276207301946195, 2.1522576752281716e-06)
_INV2PI = 0.15915494309189535
_TWOPI = 6.283185307179586
_MAGIC = 1.5 * 2.0 ** 23


def _sin_u(z, s1, b1):
    u = z * s1 + b1
    k = jnp.floor(u)
    d = u - k
    r = d * _TWOPI - np.float32(np.pi)
    r2 = r * r
    p = _SIN_C[4]
    for i in range(3, -1, -1):
        p = p * r2 + _SIN_C[i]
    return p * r


def _fused_kernel(x_ref, RW1_ref, Rb1_ref, RW2_ref, Rb2_ref,
                  W0_ref, b0_ref, W1_ref, b1_ref, W2_ref, b2_ref,
                  W3_ref, b3_ref, W4_ref, b4_ref, W5_ref, b5_ref,
                  psc_ref, pph_ref, S_ref, out_ref):
    x = x_ref[...]
    xt = jnp.concatenate([x] * (2 * NUM_ENC), axis=-1)
    pe = _sin_u(xt, psc_ref[...], pph_ref[...])
    enc = jnp.concatenate([x, pe], axis=-1)

    hr = jnp.dot(enc, RW1_ref[...], preferred_element_type=jnp.float32)
    hr = jnp.maximum(hr + Rb1_ref[...], 0.0)
    logits = jnp.dot(hr, RW2_ref[...], preferred_element_type=jnp.float32)
    logits = (logits + Rb2_ref[...]) * (1.0 / TEMP)
    m = jnp.max(logits, axis=-1, keepdims=True)
    ex = jnp.exp(logits - m)
    w = ex / jnp.sum(ex, axis=-1, keepdims=True)

    z0 = jnp.dot(enc, W0_ref[...], preferred_element_type=jnp.float32)
    h0 = _sin_u(z0, OMEGAS[0] * _INV2PI, b0_ref[...])
    hs = [h0[:, 256 * e:256 * (e + 1)] for e in range(6)]

    for li, W_ref, b_ref in ((1, W1_ref, b1_ref), (2, W2_ref, b2_ref),
                             (3, W3_ref, b3_ref), (4, W4_ref, b4_ref)):
        zs = [jnp.dot(hs[e], W_ref[e], preferred_element_type=jnp.float32)
              for e in range(6)]
        hs = [_sin_u(zs[e], OMEGAS[li] * _INV2PI, b_ref[e]) for e in range(6)]

    hcat = jnp.concatenate(hs, axis=-1)
    z18 = jnp.dot(hcat, W5_ref[...], preferred_element_type=jnp.float32)
    y18 = _sin_u(z18, OMEGAS[5] * _INV2PI, b5_ref[...])
    w3 = jnp.concatenate([w, w, w], axis=-1)
    out_ref[...] = jnp.dot(y18 * w3, S_ref[...],
                           preferred_element_type=jnp.float32)


def kernel(x, RW1, Rb1, RW2, Rb2, EW0, Eb0, EW1, Eb1, EW2, Eb2,
           EW3, Eb3, EW4, Eb4, EW5, Eb5):
    n = x.shape[0]
    tile = TILE if n % TILE == 0 else n
    grid = n // tile

    Rb1_2d = Rb1.reshape(1, -1)
    Rb2_2d = Rb2.reshape(1, -1)
    W0c = EW0.transpose(1, 0, 2).reshape(EW0.shape[1], -1)
    b0c = Eb0.reshape(1, -1) * (OMEGAS[0] * _INV2PI) + 0.5
    Wf = [EW1, EW2, EW3, EW4]
    bf = [(Eb * (om * _INV2PI) + 0.5).reshape(Eb.shape[0], 1, Eb.shape[1])
          for Eb, om in zip((Eb1, Eb2, Eb3, Eb4), OMEGAS[1:5])]
    W5bd = (EW5[:, :, :, None] * jnp.eye(6, dtype=jnp.float32)[:, None, None, :])
    W5bd = W5bd.reshape(6 * EW5.shape[1], 3 * 6)
    b5p = (Eb5 * (OMEGAS[5] * _INV2PI) + 0.5).T.reshape(1, -1)
    Ssel = np.zeros((18, 3), np.float32)
    for o in range(3):
        Ssel[6 * o:6 * o + 6, o] = 1.0
    Ssel = jnp.asarray(Ssel)

    psc = np.zeros((1, 2 * NUM_ENC * 2), np.float32)
    pph = np.zeros((1, 2 * NUM_ENC * 2), np.float32)
    for i in range(NUM_ENC):
        s = (2.0 ** i) * np.pi
        psc[0, 4 * i:4 * i + 4] = s * _INV2PI
        pph[0, 4 * i + 2:4 * i + 4] = 0.25
    pph += 0.5
    psc = jnp.asarray(psc)
    pph = jnp.asarray(pph)

    def full(a):
        nd = a.ndim
        return pl.BlockSpec(a.shape, lambda i, _nd=nd: (0,) * _nd)

    operands = (RW1, Rb1_2d, RW2, Rb2_2d, W0c, b0c,
                Wf[0], bf[0], Wf[1], bf[1], Wf[2], bf[2],
                Wf[3], bf[3], W5bd, b5p, psc, pph, Ssel)
    in_specs = [pl.BlockSpec((tile, 2), lambda i: (i, 0))]
    in_specs += [full(a) for a in operands]

    out = pl.pallas_call(
        _fused_kernel,
        grid=(grid,),
        in_specs=in_specs,
        out_specs=pl.BlockSpec((tile, 3), lambda i: (i, 0)),
        compiler_params=pltpu.CompilerParams(
            dimension_semantics=("arbitrary",),
            vmem_limit_bytes=128 * 1024 * 1024),
        out_shape=jax.ShapeDtypeStruct((n, 3), jnp.float32),
    )(x, *operands)
    return out
